# baseline (speedup 1.0000x reference)
import functools

import jax
import jax.numpy as jnp
import numpy as np
from jax import lax
from jax.experimental import pallas as pl
from jax.experimental.pallas import tpu as pltpu
from jax.experimental.pallas import tpu_sc as plsc

VOCAB = 1000000
HIDDEN = 64
BATCH = 16384
SEQ = 200

NC = 2
NS = 16
NW = NC * NS

ROWS_PER_TILE = BATCH // NW
CB = 4
IDS_PER_CHUNK = CB * SEQ
NCHUNK = ROWS_PER_TILE // CB
SEG = 80
NSEG = IDS_PER_CHUNK // SEG
UNROLL = 4

_PERM = ([2 * p for p in range(16)] + [2 * p + 1 for p in range(16)]
         + [32 + 2 * p for p in range(16)] + [33 + 2 * p for p in range(16)])


def _pack_bf16_pairs(table):
    u = lax.bitcast_convert_type(table, jnp.int32)
    rb = (u + jnp.int32(0x7FFF) + ((u >> 16) & jnp.int32(1))) >> 16
    ev = rb[:, 0::2] & jnp.int32(0xFFFF)
    od = rb[:, 1::2] << 16
    return ev | od


def _sc_pooled_sums(ids_flat, table):
    mesh = plsc.VectorSubcoreMesh(core_axis_name="c", subcore_axis_name="s")

    @functools.partial(
        pl.kernel,
        mesh=mesh,
        compiler_params=pltpu.CompilerParams(use_tc_tiling_on_sc=False),
        out_type=jax.ShapeDtypeStruct((BATCH, HIDDEN), jnp.float32),
        scratch_types=[
            pltpu.VMEM((2, IDS_PER_CHUNK), jnp.int32),
            pltpu.VMEM((2, IDS_PER_CHUNK, HIDDEN // 2), jnp.int32),
            pltpu.VMEM((2, CB, HIDDEN), jnp.float32),
            pltpu.SemaphoreType.DMA,
            pltpu.SemaphoreType.DMA,
            pltpu.SemaphoreType.DMA,
            pltpu.SemaphoreType.DMA,
        ],
    )
    def k(ids_hbm, table_hbm, out_hbm,
          ids_v, rows_v, sums_v, sem_g0, sem_g1, sem_i, sem_o):
        cid = lax.axis_index("c")
        sid = lax.axis_index("s")
        wid = sid * NC + cid

        id0 = wid * (NCHUNK * IDS_PER_CHUNK)
        row0 = wid * ROWS_PER_TILE
        sem_g = (sem_g0, sem_g1)

        def ids_fire(c, b):
            pltpu.async_copy(
                ids_hbm.at[pl.ds(id0 + c * IDS_PER_CHUNK, IDS_PER_CHUNK)],
                ids_v.at[b], sem_i)

        def ids_wait(b):
            pltpu.make_async_copy(ids_hbm.at[pl.ds(0, IDS_PER_CHUNK)],
                                  ids_v.at[b], sem_i).wait()

        def gather_fire(b):
            for s in range(NSEG):
                pltpu.async_copy(
                    table_hbm.at[ids_v.at[b, pl.ds(s * SEG, SEG)]],
                    rows_v.at[b, pl.ds(s * SEG, SEG)],
                    sem_g[b])

        def gather_wait(b):
            pltpu.make_async_copy(table_hbm.at[pl.ds(0, IDS_PER_CHUNK)],
                                  rows_v.at[b], sem_g[b]).wait()

        def out_wait(b):
            pltpu.make_async_copy(sums_v.at[b],
                                  out_hbm.at[pl.ds(0, CB)], sem_o).wait()

        def reduce_and_out(c, b):
            rv = rows_v.at[b]
            sv = sums_v.at[b]
            for r in range(CB):
                def tok(t, acc):
                    i0 = r * SEQ + t * UNROLL
                    out = list(acc)
                    for u in range(UNROLL):
                        for g in range(2):
                            y = rv[i0 + u, pl.ds(g * 16, 16)]
                            ev = lax.bitcast_convert_type(y << 16,
                                                          jnp.float32)
                            od = lax.bitcast_convert_type(
                                y & jnp.int32(-65536), jnp.float32)
                            out[2 * g] = out[2 * g] + ev
                            out[2 * g + 1] = out[2 * g + 1] + od
                    return tuple(out)
                acc0 = tuple(jnp.zeros((16,), jnp.float32) for _ in range(4))
                acc = lax.fori_loop(0, SEQ // UNROLL, tok, acc0)
                for q in range(4):
                    sv[r, pl.ds(q * 16, 16)] = acc[q]
            pltpu.async_copy(sv, out_hbm.at[pl.ds(row0 + c * CB, CB)], sem_o)

        ids_fire(0, 0)
        ids_wait(0)
        gather_fire(0)
        ids_fire(1, 1)

        def step(kk, carry):
            for b in range(2):
                c = 2 * kk + b
                gather_wait(b)

                @pl.when(c + 1 < NCHUNK)
                def _():
                    ids_wait(1 - b)
                    gather_fire(1 - b)

                @pl.when(c + 2 < NCHUNK)
                def _():
                    ids_fire(c + 2, b)

                @pl.when(c >= 2)
                def _():
                    out_wait(b)

                reduce_and_out(c, b)
            return carry

        lax.fori_loop(0, NCHUNK // 2, step, 0)
        out_wait(0)
        out_wait(1)

    return k(ids_flat, table)


def _tc_pooler(sums, a, b):
    bt = 512

    def body(x_ref, a_ref, b_ref, o_ref):
        o_ref[...] = jnp.dot(x_ref[...], a_ref[...],
                             preferred_element_type=jnp.float32) + b_ref[...]

    return pl.pallas_call(
        body,
        grid=(BATCH // bt,),
        in_specs=[
            pl.BlockSpec((bt, HIDDEN), lambda i: (i, 0)),
            pl.BlockSpec((HIDDEN, HIDDEN), lambda i: (0, 0)),
            pl.BlockSpec((1, HIDDEN), lambda i: (0, 0)),
        ],
        out_specs=pl.BlockSpec((bt, HIDDEN), lambda i: (i, 0)),
        out_shape=jax.ShapeDtypeStruct((BATCH, HIDDEN), jnp.float32),
    )(sums, a, b)


def kernel(input_ids, embedding_table, pooler_w, pooler_b):
    ids_flat = jnp.reshape(input_ids.astype(jnp.int32), (BATCH * SEQ,))
    table = _pack_bf16_pairs(embedding_table)
    sums = _sc_pooled_sums(ids_flat, table)
    a = (pooler_w.T * (1.0 / SEQ))[np.array(_PERM), :]
    b2d = jnp.reshape(pooler_b, (1, HIDDEN))
    return _tc_pooler(sums, a, b2d)


# speedup vs baseline: 10.5624x; 10.5624x over previous
import functools

import jax
import jax.numpy as jnp
from jax import lax
from jax.experimental import pallas as pl
from jax.experimental.pallas import tpu as pltpu
from jax.experimental.pallas import tpu_sc as plsc

VOCAB = 1000000
HIDDEN = 64
BATCH = 16384
SEQ = 200
NG = HIDDEN // 16

NC = 2
NS = 16
NW = NC * NS

ROWS_PER_TILE = BATCH // NW
CB = 4
IDS_PER_CHUNK = CB * SEQ
NCHUNK = ROWS_PER_TILE // CB
SEG = 80
NSEG = IDS_PER_CHUNK // SEG
UNROLL = 8


def _sc_pooled_sums(ids_flat, table):
    mesh = plsc.VectorSubcoreMesh(core_axis_name="c", subcore_axis_name="s")

    @functools.partial(
        pl.kernel,
        mesh=mesh,
        compiler_params=pltpu.CompilerParams(use_tc_tiling_on_sc=False),
        out_type=jax.ShapeDtypeStruct((BATCH, HIDDEN), jnp.float32),
        scratch_types=[
            pltpu.VMEM((2, IDS_PER_CHUNK), jnp.int32),
            pltpu.VMEM((2, IDS_PER_CHUNK, HIDDEN), jnp.float32),
            pltpu.VMEM((2, CB, HIDDEN), jnp.float32),
            pltpu.SemaphoreType.DMA,
            pltpu.SemaphoreType.DMA,
            pltpu.SemaphoreType.DMA,
            pltpu.SemaphoreType.DMA,
        ],
    )
    def k(ids_hbm, table_hbm, out_hbm,
          ids_v, rows_v, sums_v, sem_g0, sem_g1, sem_i, sem_o):
        cid = lax.axis_index("c")
        sid = lax.axis_index("s")
        wid = sid * NC + cid

        id0 = wid * (NCHUNK * IDS_PER_CHUNK)
        row0 = wid * ROWS_PER_TILE
        sem_g = (sem_g0, sem_g1)

        def ids_fire(c, b):
            pltpu.async_copy(
                ids_hbm.at[pl.ds(id0 + c * IDS_PER_CHUNK, IDS_PER_CHUNK)],
                ids_v.at[b], sem_i)

        def ids_wait(b):
            pltpu.make_async_copy(ids_hbm.at[pl.ds(0, IDS_PER_CHUNK)],
                                  ids_v.at[b], sem_i).wait()

        def gather_fire(b):
            for s in range(NSEG):
                pltpu.async_copy(
                    table_hbm.at[ids_v.at[b, pl.ds(s * SEG, SEG)]],
                    rows_v.at[b, pl.ds(s * SEG, SEG)],
                    sem_g[b])

        def gather_wait(b):
            pltpu.make_async_copy(table_hbm.at[pl.ds(0, IDS_PER_CHUNK)],
                                  rows_v.at[b], sem_g[b]).wait()

        def out_wait(b):
            pltpu.make_async_copy(sums_v.at[b],
                                  out_hbm.at[pl.ds(0, CB)], sem_o).wait()

        def reduce_and_out(c, b):
            rv = rows_v.at[b]
            sv = sums_v.at[b]
            for r in range(CB):
                def tok(t, acc):
                    i0 = r * SEQ + t * UNROLL
                    out = []
                    for g in range(NG):
                        sl = pl.ds(g * 16, 16)
                        vals = [rv[i0 + u, sl] for u in range(UNROLL)]
                        while len(vals) > 1:
                            vals = [vals[i] + vals[i + 1]
                                    for i in range(0, len(vals), 2)]
                        out.append(acc[g] + vals[0])
                    return tuple(out)
                acc0 = tuple(jnp.zeros((16,), jnp.float32)
                             for _ in range(NG))
                acc = lax.fori_loop(0, SEQ // UNROLL, tok, acc0)
                for g in range(NG):
                    sv[r, pl.ds(g * 16, 16)] = acc[g]
            pltpu.async_copy(sv, out_hbm.at[pl.ds(row0 + c * CB, CB)], sem_o)

        ids_fire(0, 0)
        ids_wait(0)
        gather_fire(0)
        ids_fire(1, 1)

        def step(kk, carry):
            for b in range(2):
                c = 2 * kk + b
                gather_wait(b)

                @pl.when(c + 1 < NCHUNK)
                def _():
                    ids_wait(1 - b)
                    gather_fire(1 - b)

                @pl.when(c + 2 < NCHUNK)
                def _():
                    ids_fire(c + 2, b)

                @pl.when(c >= 2)
                def _():
                    out_wait(b)

                reduce_and_out(c, b)
            return carry

        lax.fori_loop(0, NCHUNK // 2, step, 0)
        out_wait(0)
        out_wait(1)

    return k(ids_flat, table)


def _tc_pooler(sums, a, b):
    bt = 512

    def body(x_ref, a_ref, b_ref, o_ref):
        o_ref[...] = jnp.dot(x_ref[...], a_ref[...],
                             preferred_element_type=jnp.float32) + b_ref[...]

    return pl.pallas_call(
        body,
        grid=(BATCH // bt,),
        in_specs=[
            pl.BlockSpec((bt, HIDDEN), lambda i: (i, 0)),
            pl.BlockSpec((HIDDEN, HIDDEN), lambda i: (0, 0)),
            pl.BlockSpec((1, HIDDEN), lambda i: (0, 0)),
        ],
        out_specs=pl.BlockSpec((bt, HIDDEN), lambda i: (i, 0)),
        out_shape=jax.ShapeDtypeStruct((BATCH, HIDDEN), jnp.float32),
    )(sums, a, b)


def kernel(input_ids, embedding_table, pooler_w, pooler_b):
    ids_flat = jnp.reshape(input_ids.astype(jnp.int32), (BATCH * SEQ,))
    sums = _sc_pooled_sums(ids_flat, embedding_table)
    a = pooler_w.T * (1.0 / SEQ)
    b2d = jnp.reshape(pooler_b, (1, HIDDEN))
    return _tc_pooler(sums, a, b2d)
